# baseline (speedup 1.0000x reference)
import jax
import jax.numpy as jnp
from jax import lax
from jax.experimental import pallas as pl
from jax.experimental.pallas import tpu as pltpu
from jax.experimental.pallas import tpu_sc as plsc

L = 16
NC, NS = 2, 16
NW = NC * NS
C = 32
SUB = 16
GRP = 8
EPS = 1e-12


def _make_sc_kernel(B, D):
    b_per_w = B // NW
    steps = b_per_w // C
    n_sl = D // L
    n_grp = n_sl // L
    inv_d = 1.0 / D
    mesh = plsc.VectorSubcoreMesh(core_axis_name="c", subcore_axis_name="s",
                                  num_cores=NC, num_subcores=NS)

    def body(ids_hbm, w_hbm, g_hbm, b_hbm, out_hbm,
             idx_v, rows0, rows1, outb0, outb1, g_v, b_v, rbuf, cbuf,
             gs0, gs1, os0, os1):
        cid = lax.axis_index("c")
        sid = lax.axis_index("s")
        wid = sid * NC + cid
        pltpu.sync_copy(g_hbm, g_v)
        pltpu.sync_copy(b_hbm, b_v)
        pltpu.sync_copy(ids_hbm.at[wid], idx_v)

        def compute(rows_ref, out_ref):
            for s in range(C // SUB):
                @plsc.parallel_loop(0, SUB, step=1, unroll=4)
                def _srow(r, s=s):
                    rr = s * SUB + r
                    acc = jnp.zeros((L,), jnp.float32)
                    acc2 = jnp.zeros((L,), jnp.float32)
                    for j in range(n_sl):
                        v = rows_ref[rr, pl.ds(j * L, L)]
                        acc = acc + v
                        acc2 = acc2 + v * v
                    s1 = jnp.sum(acc)
                    s2 = jnp.sum(acc2)
                    mean = lax.broadcast_in_dim(s1, (L,), ()) * inv_d
                    ex2 = lax.broadcast_in_dim(s2, (L,), ()) * inv_d
                    x = ex2 - mean * mean + EPS
                    seed = 0x5F3759DF - (
                        lax.bitcast_convert_type(x, jnp.int32) >> 1)
                    y = lax.bitcast_convert_type(seed, jnp.float32)
                    for _ in range(2):
                        y = y * (1.5 - (0.5 * x) * (y * y))
                    rbuf[r, :] = y
                    cbuf[r, :] = mean * y

                for gi in range(n_sl // GRP):
                    gv = [g_v[pl.ds((gi * GRP + j) * L, L)]
                          for j in range(GRP)]
                    bv = [b_v[pl.ds((gi * GRP + j) * L, L)]
                          for j in range(GRP)]

                    @plsc.parallel_loop(0, SUB, step=1, unroll=4)
                    def _nrow(r, s=s, gi=gi, gv=gv, bv=bv):
                        rr = s * SUB + r
                        rv = rbuf[r, :]
                        cv = cbuf[r, :]
                        for j in range(GRP):
                            col = (gi * GRP + j) * L
                            v = rows_ref[rr, pl.ds(col, L)]
                            out_ref[rr, pl.ds(col, L)] = (
                                v * rv - cv) * gv[j] + bv[j]

        pltpu.async_copy(w_hbm.at[idx_v.at[0]], rows0, gs0)

        def dstep(h, carry):
            c0 = 2 * h
            c1 = c0 + 1
            pltpu.async_copy(w_hbm.at[idx_v.at[c1]], rows1, gs1)
            pltpu.make_async_copy(w_hbm.at[idx_v.at[c0]], rows0, gs0).wait()

            @pl.when(h > 0)
            def _():
                pltpu.make_async_copy(outb0, out_hbm.at[wid, c0 - 2],
                                      os0).wait()

            compute(rows0, outb0)
            pltpu.async_copy(outb0, out_hbm.at[wid, c0], os0)

            @pl.when(c0 + 2 < steps)
            def _():
                pltpu.async_copy(w_hbm.at[idx_v.at[c0 + 2]], rows0, gs0)

            pltpu.make_async_copy(w_hbm.at[idx_v.at[c1]], rows1, gs1).wait()

            @pl.when(h > 0)
            def _():
                pltpu.make_async_copy(outb1, out_hbm.at[wid, c1 - 2],
                                      os1).wait()

            compute(rows1, outb1)
            pltpu.async_copy(outb1, out_hbm.at[wid, c1], os1)
            return carry

        lax.fori_loop(0, steps // 2, dstep, 0)
        pltpu.make_async_copy(outb0, out_hbm.at[wid, steps - 2], os0).wait()
        pltpu.make_async_copy(outb1, out_hbm.at[wid, steps - 1], os1).wait()

    return pl.kernel(
        body,
        out_type=jax.ShapeDtypeStruct((NW, steps, C, D), jnp.float32),
        mesh=mesh,
        compiler_params=pltpu.CompilerParams(needs_layout_passes=False),
        scratch_types=[
            pltpu.VMEM((steps, C), jnp.int32),
            pltpu.VMEM((C, D), jnp.float32),
            pltpu.VMEM((C, D), jnp.float32),
            pltpu.VMEM((C, D), jnp.float32),
            pltpu.VMEM((C, D), jnp.float32),
            pltpu.VMEM((D,), jnp.float32),
            pltpu.VMEM((D,), jnp.float32),
            pltpu.VMEM((SUB, L), jnp.float32),
            pltpu.VMEM((SUB, L), jnp.float32),
            pltpu.SemaphoreType.DMA,
            pltpu.SemaphoreType.DMA,
            pltpu.SemaphoreType.DMA,
            pltpu.SemaphoreType.DMA,
        ],
    )


def kernel(input_ids, W, gamma, beta):
    orig_shape = input_ids.shape
    B = input_ids.size
    _, D = W.shape
    ids = input_ids.reshape(NW, B // NW // C, C).astype(jnp.int32)
    out = _make_sc_kernel(B, D)(ids, W, gamma, beta)
    return out.reshape(*orig_shape, D)


# speedup vs baseline: 1.2744x; 1.2744x over previous
import functools

import jax
import jax.numpy as jnp
from jax import lax
from jax.experimental import pallas as pl
from jax.experimental.pallas import tpu as pltpu
from jax.experimental.pallas import tpu_sc as plsc

L = 16
NC, NS = 2, 16
NW = NC * NS
C = 32
NBUF = 4
EPS = 1e-12
N_SLICES = 1
TC_BLK = 256


def _make_sc_gather(steps, D):
    mesh = plsc.VectorSubcoreMesh(core_axis_name="c", subcore_axis_name="s",
                                  num_cores=NC, num_subcores=NS)

    def body(ids_hbm, w_hbm, out_hbm, idx_v, b0, b1, b2, b3,
             g0, g1, g2, g3, o0, o1, o2, o3):
        cid = lax.axis_index("c")
        sid = lax.axis_index("s")
        wid = sid * NC + cid
        pltpu.sync_copy(ids_hbm.at[wid], idx_v)

        bufs = (b0, b1, b2, b3)
        gsems = (g0, g1, g2, g3)
        osems = (o0, o1, o2, o3)

        def gather(c, k):
            pltpu.async_copy(w_hbm.at[idx_v.at[c]], bufs[k], gsems[k])

        def phase(c, k):
            pltpu.make_async_copy(w_hbm.at[idx_v.at[c]], bufs[k],
                                  gsems[k]).wait()

            @pl.when(c >= 2)
            def _():
                k2 = (k + 2) % NBUF
                pltpu.make_async_copy(bufs[k2], out_hbm.at[wid, c - 2],
                                      osems[k2]).wait()

            pltpu.async_copy(bufs[k], out_hbm.at[wid, c], osems[k])

            @pl.when(c + 2 < steps)
            def _():
                gather(c + 2, (k + 2) % NBUF)

        gather(0, 0)
        gather(1, 1)

        def dstep(h, carry):
            c0 = NBUF * h
            for k in range(NBUF):
                phase(c0 + k, k)
            return carry

        lax.fori_loop(0, steps // NBUF, dstep, 0)
        pltpu.make_async_copy(bufs[(steps - 2) % NBUF],
                              out_hbm.at[wid, steps - 2],
                              osems[(steps - 2) % NBUF]).wait()
        pltpu.make_async_copy(bufs[(steps - 1) % NBUF],
                              out_hbm.at[wid, steps - 1],
                              osems[(steps - 1) % NBUF]).wait()

    return pl.kernel(
        body,
        out_type=jax.ShapeDtypeStruct((NW, steps, C, D), jnp.float32),
        mesh=mesh,
        compiler_params=pltpu.CompilerParams(needs_layout_passes=False),
        scratch_types=(
            [pltpu.VMEM((steps, C), jnp.int32)]
            + [pltpu.VMEM((C, D), jnp.float32)] * NBUF
            + [pltpu.SemaphoreType.DMA] * (2 * NBUF)
        ),
    )


def _tc_ln_kernel(x_ref, g_ref, b_ref, o_ref):
    v = x_ref[...]
    mean = jnp.mean(v, axis=1, keepdims=True)
    var = jnp.mean(v * v, axis=1, keepdims=True) - mean * mean
    rinv = lax.rsqrt(var + EPS)
    o_ref[...] = (v - mean) * rinv * g_ref[...] + b_ref[...]


def _make_tc_ln(R, D):
    grid = (R // TC_BLK,)
    return pl.pallas_call(
        _tc_ln_kernel,
        grid=grid,
        in_specs=[
            pl.BlockSpec((TC_BLK, D), lambda i: (i, 0)),
            pl.BlockSpec((1, D), lambda i: (0, 0)),
            pl.BlockSpec((1, D), lambda i: (0, 0)),
        ],
        out_specs=pl.BlockSpec((TC_BLK, D), lambda i: (i, 0)),
        out_shape=jax.ShapeDtypeStruct((R, D), jnp.float32),
        compiler_params=pltpu.CompilerParams(
            dimension_semantics=("arbitrary",)),
    )


def kernel(input_ids, W, gamma, beta):
    orig_shape = input_ids.shape
    B = input_ids.size
    _, D = W.shape
    b_slice = B // N_SLICES
    steps = b_slice // (NW * C)
    sc_gather = _make_sc_gather(steps, D)
    tc_ln = _make_tc_ln(b_slice, D)
    g2 = gamma.reshape(1, D)
    b2 = beta.reshape(1, D)
    ids = input_ids.reshape(N_SLICES, NW, steps, C).astype(jnp.int32)
    raws = [sc_gather(ids[si], W) for si in range(N_SLICES)]
    outs = [tc_ln(raw.reshape(b_slice, D), g2, b2) for raw in raws]
    out = jnp.concatenate(outs, axis=0)
    return out.reshape(*orig_shape, D)


# speedup vs baseline: 1.5113x; 1.1859x over previous
import functools

import jax
import jax.numpy as jnp
from jax import lax
from jax.experimental import pallas as pl
from jax.experimental.pallas import tpu as pltpu
from jax.experimental.pallas import tpu_sc as plsc

L = 16
NC, NS = 2, 16
NW = NC * NS
C = 32
NBUF = 4
EPS = 1e-12
N_SLICES = 1
TC_BLK = 512


def _make_sc_gather(steps, D):
    mesh = plsc.VectorSubcoreMesh(core_axis_name="c", subcore_axis_name="s",
                                  num_cores=NC, num_subcores=NS)

    def body(ids_hbm, w_hbm, out_hbm, idx_v, b0, b1, b2, b3,
             g0, g1, g2, g3, o0, o1, o2, o3):
        cid = lax.axis_index("c")
        sid = lax.axis_index("s")
        wid = sid * NC + cid
        pltpu.sync_copy(ids_hbm.at[wid], idx_v)

        bufs = (b0, b1, b2, b3)
        gsems = (g0, g1, g2, g3)
        osems = (o0, o1, o2, o3)

        def gather(c, k):
            pltpu.async_copy(w_hbm.at[idx_v.at[c]], bufs[k], gsems[k])

        def phase(c, k):
            pltpu.make_async_copy(w_hbm.at[idx_v.at[c]], bufs[k],
                                  gsems[k]).wait()

            @pl.when(c >= 2)
            def _():
                k2 = (k + 2) % NBUF
                pltpu.make_async_copy(bufs[k2], out_hbm.at[wid, c - 2],
                                      osems[k2]).wait()

            pltpu.async_copy(bufs[k], out_hbm.at[wid, c], osems[k])

            @pl.when(c + 2 < steps)
            def _():
                gather(c + 2, (k + 2) % NBUF)

        gather(0, 0)
        gather(1, 1)

        def dstep(h, carry):
            c0 = NBUF * h
            for k in range(NBUF):
                phase(c0 + k, k)
            return carry

        lax.fori_loop(0, steps // NBUF, dstep, 0)
        pltpu.make_async_copy(bufs[(steps - 2) % NBUF],
                              out_hbm.at[wid, steps - 2],
                              osems[(steps - 2) % NBUF]).wait()
        pltpu.make_async_copy(bufs[(steps - 1) % NBUF],
                              out_hbm.at[wid, steps - 1],
                              osems[(steps - 1) % NBUF]).wait()

    return pl.kernel(
        body,
        out_type=jax.ShapeDtypeStruct((NW, steps, C, D), jnp.float32),
        mesh=mesh,
        compiler_params=pltpu.CompilerParams(needs_layout_passes=False),
        scratch_types=(
            [pltpu.VMEM((steps, C), jnp.int32)]
            + [pltpu.VMEM((C, D), jnp.float32)] * NBUF
            + [pltpu.SemaphoreType.DMA] * (2 * NBUF)
        ),
    )


def _tc_ln_kernel(x_ref, g_ref, b_ref, o_ref):
    v = x_ref[...]
    mean = jnp.mean(v, axis=1, keepdims=True)
    var = jnp.mean(v * v, axis=1, keepdims=True) - mean * mean
    rinv = lax.rsqrt(var + EPS)
    o_ref[...] = (v - mean) * rinv * g_ref[...] + b_ref[...]


def _make_tc_ln(R, D):
    grid = (R // TC_BLK,)
    return pl.pallas_call(
        _tc_ln_kernel,
        grid=grid,
        in_specs=[
            pl.BlockSpec((TC_BLK, D), lambda i: (i, 0)),
            pl.BlockSpec((1, D), lambda i: (0, 0)),
            pl.BlockSpec((1, D), lambda i: (0, 0)),
        ],
        out_specs=pl.BlockSpec((TC_BLK, D), lambda i: (i, 0)),
        out_shape=jax.ShapeDtypeStruct((R, D), jnp.float32),
        compiler_params=pltpu.CompilerParams(
            dimension_semantics=("arbitrary",)),
    )


def kernel(input_ids, W, gamma, beta):
    orig_shape = input_ids.shape
    B = input_ids.size
    _, D = W.shape
    b_slice = B // N_SLICES
    steps = b_slice // (NW * C)
    sc_gather = _make_sc_gather(steps, D)
    tc_ln = _make_tc_ln(b_slice, D)
    g2 = gamma.reshape(1, D)
    b2 = beta.reshape(1, D)
    ids = input_ids.reshape(N_SLICES, NW, steps, C).astype(jnp.int32)
    raws = [sc_gather(ids[si], W) for si in range(N_SLICES)]
    outs = [tc_ln(raw.reshape(b_slice, D), g2, b2) for raw in raws]
    out = jnp.concatenate(outs, axis=0)
    return out.reshape(*orig_shape, D)


# speedup vs baseline: 1.6677x; 1.1036x over previous
import functools

import jax
import jax.numpy as jnp
from jax import lax
from jax.experimental import pallas as pl
from jax.experimental.pallas import tpu as pltpu
from jax.experimental.pallas import tpu_sc as plsc

L = 16
NC, NS = 2, 16
NW = NC * NS
C = 32
NBUF = 4
EPS = 1e-12
N_SLICES = 1
TC_BLK = 1024


def _make_sc_gather(steps, D):
    mesh = plsc.VectorSubcoreMesh(core_axis_name="c", subcore_axis_name="s",
                                  num_cores=NC, num_subcores=NS)

    def body(ids_hbm, w_hbm, out_hbm, idx_v, b0, b1, b2, b3,
             g0, g1, g2, g3, o0, o1, o2, o3):
        cid = lax.axis_index("c")
        sid = lax.axis_index("s")
        wid = sid * NC + cid
        pltpu.sync_copy(ids_hbm.at[wid], idx_v)

        bufs = (b0, b1, b2, b3)
        gsems = (g0, g1, g2, g3)
        osems = (o0, o1, o2, o3)

        def gather(c, k):
            pltpu.async_copy(w_hbm.at[idx_v.at[c]], bufs[k], gsems[k])

        def phase(c, k):
            pltpu.make_async_copy(w_hbm.at[idx_v.at[c]], bufs[k],
                                  gsems[k]).wait()

            @pl.when(c >= 2)
            def _():
                k2 = (k + 2) % NBUF
                pltpu.make_async_copy(bufs[k2], out_hbm.at[wid, c - 2],
                                      osems[k2]).wait()

            pltpu.async_copy(bufs[k], out_hbm.at[wid, c], osems[k])

            @pl.when(c + 2 < steps)
            def _():
                gather(c + 2, (k + 2) % NBUF)

        gather(0, 0)
        gather(1, 1)

        def dstep(h, carry):
            c0 = NBUF * h
            for k in range(NBUF):
                phase(c0 + k, k)
            return carry

        lax.fori_loop(0, steps // NBUF, dstep, 0)
        pltpu.make_async_copy(bufs[(steps - 2) % NBUF],
                              out_hbm.at[wid, steps - 2],
                              osems[(steps - 2) % NBUF]).wait()
        pltpu.make_async_copy(bufs[(steps - 1) % NBUF],
                              out_hbm.at[wid, steps - 1],
                              osems[(steps - 1) % NBUF]).wait()

    return pl.kernel(
        body,
        out_type=jax.ShapeDtypeStruct((NW, steps, C, D), jnp.float32),
        mesh=mesh,
        compiler_params=pltpu.CompilerParams(needs_layout_passes=False),
        scratch_types=(
            [pltpu.VMEM((steps, C), jnp.int32)]
            + [pltpu.VMEM((C, D), jnp.float32)] * NBUF
            + [pltpu.SemaphoreType.DMA] * (2 * NBUF)
        ),
    )


def _tc_ln_kernel(x_ref, g_ref, b_ref, o_ref):
    v = x_ref[...]
    mean = jnp.mean(v, axis=1, keepdims=True)
    var = jnp.mean(v * v, axis=1, keepdims=True) - mean * mean
    rinv = lax.rsqrt(var + EPS)
    o_ref[...] = (v - mean) * rinv * g_ref[...] + b_ref[...]


def _make_tc_ln(R, D):
    grid = (R // TC_BLK,)
    return pl.pallas_call(
        _tc_ln_kernel,
        grid=grid,
        in_specs=[
            pl.BlockSpec((TC_BLK, D), lambda i: (i, 0)),
            pl.BlockSpec((1, D), lambda i: (0, 0)),
            pl.BlockSpec((1, D), lambda i: (0, 0)),
        ],
        out_specs=pl.BlockSpec((TC_BLK, D), lambda i: (i, 0)),
        out_shape=jax.ShapeDtypeStruct((R, D), jnp.float32),
        compiler_params=pltpu.CompilerParams(
            dimension_semantics=("arbitrary",)),
    )


def kernel(input_ids, W, gamma, beta):
    orig_shape = input_ids.shape
    B = input_ids.size
    _, D = W.shape
    b_slice = B // N_SLICES
    steps = b_slice // (NW * C)
    sc_gather = _make_sc_gather(steps, D)
    tc_ln = _make_tc_ln(b_slice, D)
    g2 = gamma.reshape(1, D)
    b2 = beta.reshape(1, D)
    ids = input_ids.reshape(N_SLICES, NW, steps, C).astype(jnp.int32)
    raws = [sc_gather(ids[si], W) for si in range(N_SLICES)]
    outs = [tc_ln(raw.reshape(b_slice, D), g2, b2) for raw in raws]
    out = jnp.concatenate(outs, axis=0)
    return out.reshape(*orig_shape, D)


# speedup vs baseline: 1.7190x; 1.0307x over previous
import functools

import jax
import jax.numpy as jnp
from jax import lax
from jax.experimental import pallas as pl
from jax.experimental.pallas import tpu as pltpu
from jax.experimental.pallas import tpu_sc as plsc

L = 16
NC, NS = 2, 16
NW = NC * NS
C = 32
NBUF = 4
EPS = 1e-12
N_SLICES = 1
TC_BLK = 2048


def _make_sc_gather(steps, D):
    mesh = plsc.VectorSubcoreMesh(core_axis_name="c", subcore_axis_name="s",
                                  num_cores=NC, num_subcores=NS)

    def body(ids_hbm, w_hbm, out_hbm, idx_v, b0, b1, b2, b3,
             g0, g1, g2, g3, o0, o1, o2, o3):
        cid = lax.axis_index("c")
        sid = lax.axis_index("s")
        wid = sid * NC + cid
        pltpu.sync_copy(ids_hbm.at[wid], idx_v)

        bufs = (b0, b1, b2, b3)
        gsems = (g0, g1, g2, g3)
        osems = (o0, o1, o2, o3)

        def gather(c, k):
            pltpu.async_copy(w_hbm.at[idx_v.at[c]], bufs[k], gsems[k])

        def phase(c, k):
            pltpu.make_async_copy(w_hbm.at[idx_v.at[c]], bufs[k],
                                  gsems[k]).wait()

            @pl.when(c >= 2)
            def _():
                k2 = (k + 2) % NBUF
                pltpu.make_async_copy(bufs[k2], out_hbm.at[wid, c - 2],
                                      osems[k2]).wait()

            pltpu.async_copy(bufs[k], out_hbm.at[wid, c], osems[k])

            @pl.when(c + 2 < steps)
            def _():
                gather(c + 2, (k + 2) % NBUF)

        gather(0, 0)
        gather(1, 1)

        def dstep(h, carry):
            c0 = NBUF * h
            for k in range(NBUF):
                phase(c0 + k, k)
            return carry

        lax.fori_loop(0, steps // NBUF, dstep, 0)
        pltpu.make_async_copy(bufs[(steps - 2) % NBUF],
                              out_hbm.at[wid, steps - 2],
                              osems[(steps - 2) % NBUF]).wait()
        pltpu.make_async_copy(bufs[(steps - 1) % NBUF],
                              out_hbm.at[wid, steps - 1],
                              osems[(steps - 1) % NBUF]).wait()

    return pl.kernel(
        body,
        out_type=jax.ShapeDtypeStruct((NW, steps, C, D), jnp.float32),
        mesh=mesh,
        compiler_params=pltpu.CompilerParams(needs_layout_passes=False),
        scratch_types=(
            [pltpu.VMEM((steps, C), jnp.int32)]
            + [pltpu.VMEM((C, D), jnp.float32)] * NBUF
            + [pltpu.SemaphoreType.DMA] * (2 * NBUF)
        ),
    )


def _tc_ln_kernel(x_ref, g_ref, b_ref, o_ref):
    v = x_ref[...]
    mean = jnp.mean(v, axis=1, keepdims=True)
    var = jnp.mean(v * v, axis=1, keepdims=True) - mean * mean
    rinv = lax.rsqrt(var + EPS)
    o_ref[...] = (v - mean) * rinv * g_ref[...] + b_ref[...]


def _make_tc_ln(R, D):
    grid = (R // TC_BLK,)
    return pl.pallas_call(
        _tc_ln_kernel,
        grid=grid,
        in_specs=[
            pl.BlockSpec((TC_BLK, D), lambda i: (i, 0)),
            pl.BlockSpec((1, D), lambda i: (0, 0)),
            pl.BlockSpec((1, D), lambda i: (0, 0)),
        ],
        out_specs=pl.BlockSpec((TC_BLK, D), lambda i: (i, 0)),
        out_shape=jax.ShapeDtypeStruct((R, D), jnp.float32),
        compiler_params=pltpu.CompilerParams(
            dimension_semantics=("arbitrary",)),
    )


def kernel(input_ids, W, gamma, beta):
    orig_shape = input_ids.shape
    B = input_ids.size
    _, D = W.shape
    b_slice = B // N_SLICES
    steps = b_slice // (NW * C)
    sc_gather = _make_sc_gather(steps, D)
    tc_ln = _make_tc_ln(b_slice, D)
    g2 = gamma.reshape(1, D)
    b2 = beta.reshape(1, D)
    ids = input_ids.reshape(N_SLICES, NW, steps, C).astype(jnp.int32)
    raws = [sc_gather(ids[si], W) for si in range(N_SLICES)]
    outs = [tc_ln(raw.reshape(b_slice, D), g2, b2) for raw in raws]
    out = jnp.concatenate(outs, axis=0)
    return out.reshape(*orig_shape, D)


# speedup vs baseline: 1.7390x; 1.0117x over previous
import functools

import jax
import jax.numpy as jnp
from jax import lax
from jax.experimental import pallas as pl
from jax.experimental.pallas import tpu as pltpu
from jax.experimental.pallas import tpu_sc as plsc

L = 16
NC, NS = 2, 16
NW = NC * NS
C = 32
NBUF = 4
EPS = 1e-12
N_SLICES = 1
TC_BLK = 2048


def _make_sc_gather(steps, D):
    mesh = plsc.VectorSubcoreMesh(core_axis_name="c", subcore_axis_name="s",
                                  num_cores=NC, num_subcores=NS)

    def body(ids_hbm, w_hbm, out_hbm, idx_v, b0, b1, b2, b3,
             g0, g1, g2, g3, o0, o1, o2, o3):
        cid = lax.axis_index("c")
        sid = lax.axis_index("s")
        wid = sid * NC + cid
        pltpu.sync_copy(ids_hbm.at[wid], idx_v)

        bufs = (b0, b1, b2, b3)
        gsems = (g0, g1, g2, g3)
        osems = (o0, o1, o2, o3)

        def gather(c, k):
            pltpu.async_copy(w_hbm.at[idx_v.at[c]], bufs[k], gsems[k])

        def phase(c, k):
            pltpu.make_async_copy(w_hbm.at[idx_v.at[c]], bufs[k],
                                  gsems[k]).wait()
            pltpu.async_copy(bufs[k], out_hbm.at[wid, c], osems[k])

            @pl.when(c >= 1)
            def _():
                k3 = (k + 3) % NBUF
                pltpu.make_async_copy(bufs[k3], out_hbm.at[wid, c - 1],
                                      osems[k3]).wait()

            @pl.when(c + 3 < steps)
            def _():
                gather(c + 3, (k + 3) % NBUF)

        gather(0, 0)
        gather(1, 1)
        gather(2, 2)

        def dstep(h, carry):
            c0 = NBUF * h
            for k in range(NBUF):
                phase(c0 + k, k)
            return carry

        lax.fori_loop(0, steps // NBUF, dstep, 0)
        pltpu.make_async_copy(bufs[(steps - 1) % NBUF],
                              out_hbm.at[wid, steps - 1],
                              osems[(steps - 1) % NBUF]).wait()

    return pl.kernel(
        body,
        out_type=jax.ShapeDtypeStruct((NW, steps, C, D), jnp.float32),
        mesh=mesh,
        compiler_params=pltpu.CompilerParams(needs_layout_passes=False),
        scratch_types=(
            [pltpu.VMEM((steps, C), jnp.int32)]
            + [pltpu.VMEM((C, D), jnp.float32)] * NBUF
            + [pltpu.SemaphoreType.DMA] * (2 * NBUF)
        ),
    )


def _tc_ln_kernel(x_ref, g_ref, b_ref, o_ref):
    v = x_ref[...]
    mean = jnp.mean(v, axis=1, keepdims=True)
    var = jnp.mean(v * v, axis=1, keepdims=True) - mean * mean
    rinv = lax.rsqrt(var + EPS)
    o_ref[...] = (v - mean) * rinv * g_ref[...] + b_ref[...]


def _make_tc_ln(R, D):
    grid = (R // TC_BLK,)
    return pl.pallas_call(
        _tc_ln_kernel,
        grid=grid,
        in_specs=[
            pl.BlockSpec((TC_BLK, D), lambda i: (i, 0)),
            pl.BlockSpec((1, D), lambda i: (0, 0)),
            pl.BlockSpec((1, D), lambda i: (0, 0)),
        ],
        out_specs=pl.BlockSpec((TC_BLK, D), lambda i: (i, 0)),
        out_shape=jax.ShapeDtypeStruct((R, D), jnp.float32),
        compiler_params=pltpu.CompilerParams(
            dimension_semantics=("arbitrary",)),
    )


def kernel(input_ids, W, gamma, beta):
    orig_shape = input_ids.shape
    B = input_ids.size
    _, D = W.shape
    b_slice = B // N_SLICES
    steps = b_slice // (NW * C)
    sc_gather = _make_sc_gather(steps, D)
    tc_ln = _make_tc_ln(b_slice, D)
    g2 = gamma.reshape(1, D)
    b2 = beta.reshape(1, D)
    ids = input_ids.reshape(N_SLICES, NW, steps, C).astype(jnp.int32)
    raws = [sc_gather(ids[si], W) for si in range(N_SLICES)]
    outs = [tc_ln(raw.reshape(b_slice, D), g2, b2) for raw in raws]
    out = jnp.concatenate(outs, axis=0)
    return out.reshape(*orig_shape, D)
